# Initial kernel scaffold; baseline (speedup 1.0000x reference)
import jax
import jax.numpy as jnp
from jax.experimental import pallas as pl


def kernel(input):
    raise NotImplementedError("write your pallas kernel here")


# speedup vs baseline: 13.5585x; 13.5585x over previous
import jax
import jax.numpy as jnp
from jax.experimental import pallas as pl


def _body(x_ref, o_ref):
    N = x_ref.shape[1]
    d = x_ref.shape[2]
    o_ref[...] = jnp.zeros_like(o_ref)
    for i in range(N):
        o_ref[0, i * d:(i + 1) * d, i * d:(i + 1) * d] = x_ref[0, i]


def kernel(input):
    B, N, d, _ = input.shape
    M = N * d
    return pl.pallas_call(
        _body,
        grid=(B,),
        in_specs=[pl.BlockSpec((1, N, d, d), lambda b: (b, 0, 0, 0))],
        out_specs=pl.BlockSpec((1, M, M), lambda b: (b, 0, 0)),
        out_shape=jax.ShapeDtypeStruct((B, M, M), input.dtype),
    )(input)
